# Initial kernel scaffold; baseline (speedup 1.0000x reference)
import jax
import jax.numpy as jnp
from jax.experimental import pallas as pl


def kernel(h, edge_index, e, snorm_n, snorm_e, W_emb, b_emb, Ws, bs, gammas, betas, W_mlp1, b_mlp1, W_mlp2, b_mlp2):
    raise NotImplementedError("write your pallas kernel here")


# speedup vs baseline: 5.6558x; 5.6558x over previous
import functools

import jax
import jax.numpy as jnp
from jax import lax
from jax.experimental import pallas as pl
from jax.experimental.pallas import tpu as pltpu
from jax.experimental.pallas import tpu_sc as plsc

_N = 10000
_E = 320000
_D = 128
_NC = 2
_NS = 16
_NW = _NC * _NS
_EPW = _E // _NW
_CH = 80
_NCHUNK = _EPW // _CH
_NP = 10240
_RPT = _NP // _NS
_BN = 2000
_G = _N // _BN

_f32 = jnp.float32


def _sc_degrees_body(src_hbm, dst_hbm, ones_hbm, z_hbm, outa_hbm, outb_hbm,
                     sidx, didx, ones_v, tab):
    c = lax.axis_index("c")
    s = lax.axis_index("s")
    wid = s * _NC + c
    my = pl.ds(s * _RPT, _RPT)
    pltpu.sync_copy(src_hbm.at[wid], sidx)
    pltpu.sync_copy(dst_hbm.at[wid], didx)
    pltpu.sync_copy(ones_hbm, ones_v)
    pltpu.sync_copy(z_hbm.at[my], tab.at[my])
    plsc.subcore_barrier()

    @pl.loop(0, _NCHUNK)
    def _(i):
        pltpu.sync_copy(ones_v, tab.at[sidx.at[i]], add=True)

    plsc.subcore_barrier()
    pltpu.sync_copy(tab.at[my], outa_hbm.at[pl.ds(c * _NP + s * _RPT, _RPT)])
    pltpu.sync_copy(z_hbm.at[my], tab.at[my])
    plsc.subcore_barrier()

    @pl.loop(0, _NCHUNK)
    def _(i):
        pltpu.sync_copy(ones_v, tab.at[didx.at[i]], add=True)

    plsc.subcore_barrier()
    pltpu.sync_copy(tab.at[my], outb_hbm.at[pl.ds(c * _NP + s * _RPT, _RPT)])


def _sc_aggregate_body(t_hbm, src_hbm, dst_hbm, znd_hbm, out_hbm,
                       sidx, didx, rows, acc):
    c = lax.axis_index("c")
    s = lax.axis_index("s")
    wid = s * _NC + c
    pltpu.sync_copy(src_hbm.at[wid], sidx)
    pltpu.sync_copy(dst_hbm.at[wid], didx)
    pltpu.sync_copy(znd_hbm.at[pl.ds(s * _RPT, _RPT)], acc.at[pl.ds(s * _RPT, _RPT)])
    plsc.subcore_barrier()

    @pl.loop(0, _NCHUNK)
    def _(i):
        pltpu.sync_copy(t_hbm.at[sidx.at[pl.ds(i * _CH, _CH)]], rows)
        pltpu.sync_copy(rows, acc.at[didx.at[i]], add=True)

    plsc.subcore_barrier()
    pltpu.sync_copy(acc.at[pl.ds(s * _RPT, _RPT)],
                    out_hbm.at[pl.ds(c * _NP + s * _RPT, _RPT)])


@functools.cache
def _sc_kernels():
    mesh = plsc.VectorSubcoreMesh(core_axis_name="c", subcore_axis_name="s",
                                  num_cores=_NC, num_subcores=_NS)
    degrees = pl.kernel(
        _sc_degrees_body,
        mesh=mesh,
        out_type=[jax.ShapeDtypeStruct((_NC * _NP, _D), _f32),
                  jax.ShapeDtypeStruct((_NC * _NP, _D), _f32)],
        scratch_types=[pltpu.VMEM((_NCHUNK, _CH), jnp.int32),
                       pltpu.VMEM((_NCHUNK, _CH), jnp.int32),
                       pltpu.VMEM((_CH, _D), _f32),
                       pltpu.VMEM_SHARED((_NP, _D), _f32)],
    )
    aggregate = pl.kernel(
        _sc_aggregate_body,
        mesh=mesh,
        out_type=jax.ShapeDtypeStruct((_NC * _NP, _D), _f32),
        scratch_types=[pltpu.VMEM((_EPW,), jnp.int32),
                       pltpu.VMEM((_NCHUNK, _CH), jnp.int32),
                       pltpu.VMEM((_CH, _D), _f32),
                       pltpu.VMEM_SHARED((_NP, _D), _f32)],
    )
    return degrees, aggregate


def _row_spec():
    return pl.BlockSpec((_BN, _D), lambda i: (i, 0))


def _full_spec(shape):
    return pl.BlockSpec(shape, lambda i: tuple(0 for _ in shape))


def _prep_body(da0, da1, db0, db1, sn, nout, ninsn, snb):
    deg_o = da0[:, :1] + da1[:, :1]
    deg_i = db0[:, :1] + db1[:, :1]
    no = jnp.where(deg_o > 0, lax.rsqrt(deg_o), 0.0)
    ni = jnp.where(deg_i > 0, lax.rsqrt(deg_i), 0.0)
    s = sn[...]
    nout[...] = jnp.broadcast_to(no, (_BN, _D))
    ninsn[...] = jnp.broadcast_to(ni * s, (_BN, _D))
    snb[...] = jnp.broadcast_to(s, (_BN, _D))


_prep = pl.pallas_call(
    _prep_body,
    grid=(_G,),
    in_specs=[_row_spec()] * 4 + [pl.BlockSpec((_BN, 1), lambda i: (i, 0))],
    out_specs=[_row_spec()] * 3,
    out_shape=[jax.ShapeDtypeStruct((_N, _D), _f32)] * 3,
)


def _embed_body(h, we, be, nout, w1, x, t):
    xv = jnp.dot(h[...], we[...], preferred_element_type=_f32) + be[...]
    x[...] = xv
    t[...] = jnp.dot(xv * nout[...], w1[...], preferred_element_type=_f32)


_embed = pl.pallas_call(
    _embed_body,
    grid=(_G,),
    in_specs=[_row_spec(), _full_spec((_D, _D)), _full_spec((1, _D)),
              _row_spec(), _full_spec((_D, _D))],
    out_specs=[_row_spec(), _row_spec()],
    out_shape=[jax.ShapeDtypeStruct((_N, _D), _f32)] * 2,
)


def _post_body(p0, p1, ninsn, snb, b, u, ssum, ssq):
    uv = (p0[...] + p1[...]) * ninsn[...] + b[...] * snb[...]
    u[...] = uv

    @pl.when(pl.program_id(0) == 0)
    def _():
        ssum[...] = jnp.zeros((1, _D), _f32)
        ssq[...] = jnp.zeros((1, _D), _f32)

    ssum[...] += jnp.sum(uv, axis=0, keepdims=True)
    ssq[...] += jnp.sum(uv * uv, axis=0, keepdims=True)


_post = pl.pallas_call(
    _post_body,
    grid=(_G,),
    in_specs=[_row_spec(), _row_spec(), _row_spec(), _row_spec(),
              _full_spec((1, _D))],
    out_specs=[_row_spec(), _full_spec((1, _D)), _full_spec((1, _D))],
    out_shape=[jax.ShapeDtypeStruct((_N, _D), _f32),
               jax.ShapeDtypeStruct((1, _D), _f32),
               jax.ShapeDtypeStruct((1, _D), _f32)],
)


def _bn_x(u, ssum, ssq, g, bt, xp):
    mean = ssum[...] * (1.0 / _N)
    var = ssq[...] * (1.0 / _N) - mean * mean
    rstd = lax.rsqrt(var + 1e-5)
    return jax.nn.relu((u[...] - mean) * rstd * g[...] + bt[...]) + xp[...]


def _bnnext_body(u, ssum, ssq, g, bt, xp, nout, wn, x, t):
    xv = _bn_x(u, ssum, ssq, g, bt, xp)
    x[...] = xv
    t[...] = jnp.dot(xv * nout[...], wn[...], preferred_element_type=_f32)


_bnnext = pl.pallas_call(
    _bnnext_body,
    grid=(_G,),
    in_specs=[_row_spec(), _full_spec((1, _D)), _full_spec((1, _D)),
              _full_spec((1, _D)), _full_spec((1, _D)), _row_spec(),
              _row_spec(), _full_spec((_D, _D))],
    out_specs=[_row_spec(), _row_spec()],
    out_shape=[jax.ShapeDtypeStruct((_N, _D), _f32)] * 2,
)


def _bnlast_body(u, ssum, ssq, g, bt, xp, xsum):
    xv = _bn_x(u, ssum, ssq, g, bt, xp)

    @pl.when(pl.program_id(0) == 0)
    def _():
        xsum[...] = jnp.zeros((1, _D), _f32)

    xsum[...] += jnp.sum(xv, axis=0, keepdims=True)


_bnlast = pl.pallas_call(
    _bnlast_body,
    grid=(_G,),
    in_specs=[_row_spec(), _full_spec((1, _D)), _full_spec((1, _D)),
              _full_spec((1, _D)), _full_spec((1, _D)), _row_spec()],
    out_specs=[_full_spec((1, _D))],
    out_shape=[jax.ShapeDtypeStruct((1, _D), _f32)],
)


def _head_body(xsum, w1, b1, w2, b2, o):
    hg = xsum[...] * (1.0 / _N)
    z = jax.nn.relu(jnp.dot(hg, w1[...], preferred_element_type=_f32) + b1[...])
    o[...] = jnp.dot(z, w2[...], preferred_element_type=_f32) + b2[...]


_head = pl.pallas_call(
    _head_body,
    in_specs=[pl.BlockSpec((1, _D), lambda: (0, 0)),
              pl.BlockSpec((_D, _D), lambda: (0, 0)),
              pl.BlockSpec((1, _D), lambda: (0, 0)),
              pl.BlockSpec((_D, 10), lambda: (0, 0)),
              pl.BlockSpec((1, 10), lambda: (0, 0))],
    out_specs=pl.BlockSpec((1, 10), lambda: (0, 0)),
    out_shape=jax.ShapeDtypeStruct((1, 10), _f32),
)


def kernel(h, edge_index, e, snorm_n, snorm_e, W_emb, b_emb, Ws, bs,
           gammas, betas, W_mlp1, b_mlp1, W_mlp2, b_mlp2):
    del e, snorm_e
    src3 = edge_index[0].reshape(_NW, _NCHUNK, _CH)
    dst3 = edge_index[1].reshape(_NW, _NCHUNK, _CH)
    src2 = edge_index[0].reshape(_NW, _EPW)
    onesd = jnp.ones((_CH, _D), _f32)
    znd = jnp.zeros((_NP, _D), _f32)

    _sc_degrees, _sc_aggregate = _sc_kernels()
    da, db = _sc_degrees(src3, dst3, onesd, znd)
    nout, ninsn, snb = _prep(da[:_N], da[_NP:_NP + _N], db[:_N], db[_NP:_NP + _N], snorm_n)

    x, t = _embed(h, W_emb, b_emb.reshape(1, _D), nout, Ws[0])
    n_layers = len(Ws)
    for i in range(n_layers):
        p = _sc_aggregate(t, src2, dst3, znd)
        u, ssum, ssq = _post(p[:_N], p[_NP:_NP + _N], ninsn, snb, bs[i].reshape(1, _D))
        g2 = gammas[i].reshape(1, _D)
        bt2 = betas[i].reshape(1, _D)
        if i < n_layers - 1:
            x, t = _bnnext(u, ssum, ssq, g2, bt2, x, nout, Ws[i + 1])
        else:
            (xsum,) = _bnlast(u, ssum, ssq, g2, bt2, x)

    return _head(xsum, W_mlp1, b_mlp1.reshape(1, _D),
                 W_mlp2, b_mlp2.reshape(1, 10))


# speedup vs baseline: 8.1451x; 1.4401x over previous
import functools

import jax
import jax.numpy as jnp
from jax import lax
from jax.experimental import pallas as pl
from jax.experimental.pallas import tpu as pltpu
from jax.experimental.pallas import tpu_sc as plsc

_N = 10000
_E = 320000
_D = 128
_NC = 2
_NS = 16
_NW = _NC * _NS
_EPW = _E // _NW
_CH = 80
_NCHUNK = _EPW // _CH
_NP = 10240
_RPT = _NP // _NS
_BN = 2000
_G = _N // _BN

_f32 = jnp.float32


def _sc_degrees_body(src_hbm, dst_hbm, ones_hbm, z_hbm, outa_hbm, outb_hbm,
                     sidx, didx, ones_v, tab):
    c = lax.axis_index("c")
    s = lax.axis_index("s")
    wid = s * _NC + c
    my = pl.ds(s * _RPT, _RPT)
    pltpu.sync_copy(src_hbm.at[wid], sidx)
    pltpu.sync_copy(dst_hbm.at[wid], didx)
    pltpu.sync_copy(ones_hbm, ones_v)
    pltpu.sync_copy(z_hbm.at[my], tab.at[my])
    plsc.subcore_barrier()

    @pl.loop(0, _NCHUNK)
    def _(i):
        pltpu.sync_copy(ones_v, tab.at[sidx.at[i]], add=True)

    plsc.subcore_barrier()
    pltpu.sync_copy(tab.at[my], outa_hbm.at[pl.ds(c * _NP + s * _RPT, _RPT)])
    pltpu.sync_copy(z_hbm.at[my], tab.at[my])
    plsc.subcore_barrier()

    @pl.loop(0, _NCHUNK)
    def _(i):
        pltpu.sync_copy(ones_v, tab.at[didx.at[i]], add=True)

    plsc.subcore_barrier()
    pltpu.sync_copy(tab.at[my], outb_hbm.at[pl.ds(c * _NP + s * _RPT, _RPT)])


def _sc_aggregate_body(t_hbm, src_hbm, dst_hbm, znd_hbm, out_hbm,
                       sidx, didx, r0, r1, acc, sg0, sg1):
    c = lax.axis_index("c")
    s = lax.axis_index("s")
    wid = s * _NC + c
    my = pl.ds(s * _RPT, _RPT)
    pltpu.sync_copy(src_hbm.at[wid], sidx)
    pltpu.sync_copy(dst_hbm.at[wid], didx)
    pltpu.sync_copy(znd_hbm.at[my], acc.at[my])
    plsc.subcore_barrier()

    def gstart(i, buf, sem):
        pltpu.async_copy(t_hbm.at[sidx.at[pl.ds(i * _CH, _CH)]], buf, sem)

    def gwait(buf, sem):
        pltpu.make_async_copy(t_hbm.at[pl.ds(0, _CH)], buf, sem).wait()

    def sadd(i, buf):
        pltpu.sync_copy(buf, acc.at[didx.at[i]], add=True)

    gstart(0, r0, sg0)
    gstart(1, r1, sg1)

    @pl.loop(0, (_NCHUNK - 3) // 2)
    def _(j):
        i = 2 * j
        gwait(r0, sg0)
        sadd(i, r0)
        gstart(i + 2, r0, sg0)
        gwait(r1, sg1)
        sadd(i + 1, r1)
        gstart(i + 3, r1, sg1)

    gwait(r0, sg0)
    sadd(_NCHUNK - 3, r0)
    gstart(_NCHUNK - 1, r0, sg0)
    gwait(r1, sg1)
    sadd(_NCHUNK - 2, r1)
    gwait(r0, sg0)
    sadd(_NCHUNK - 1, r0)

    plsc.subcore_barrier()
    pltpu.sync_copy(acc.at[my], out_hbm.at[pl.ds(c * _NP + s * _RPT, _RPT)])


@functools.cache
def _sc_kernels():
    mesh = plsc.VectorSubcoreMesh(core_axis_name="c", subcore_axis_name="s",
                                  num_cores=_NC, num_subcores=_NS)
    degrees = pl.kernel(
        _sc_degrees_body,
        mesh=mesh,
        out_type=[jax.ShapeDtypeStruct((_NC * _NP, _D), _f32),
                  jax.ShapeDtypeStruct((_NC * _NP, _D), _f32)],
        scratch_types=[pltpu.VMEM((_NCHUNK, _CH), jnp.int32),
                       pltpu.VMEM((_NCHUNK, _CH), jnp.int32),
                       pltpu.VMEM((_CH, _D), _f32),
                       pltpu.VMEM_SHARED((_NP, _D), _f32)],
    )
    aggregate = pl.kernel(
        _sc_aggregate_body,
        mesh=mesh,
        out_type=jax.ShapeDtypeStruct((_NC * _NP, _D), _f32),
        scratch_types=[pltpu.VMEM((_EPW,), jnp.int32),
                       pltpu.VMEM((_NCHUNK, _CH), jnp.int32),
                       pltpu.VMEM((_CH, _D), _f32),
                       pltpu.VMEM((_CH, _D), _f32),
                       pltpu.VMEM_SHARED((_NP, _D), _f32),
                       pltpu.SemaphoreType.DMA,
                       pltpu.SemaphoreType.DMA],
    )
    return degrees, aggregate


def _row_spec():
    return pl.BlockSpec((_BN, _D), lambda i: (i, 0))


def _full_spec(shape):
    return pl.BlockSpec(shape, lambda i: tuple(0 for _ in shape))


def _prep_body(da0, da1, db0, db1, sn, nout, ninsn, snb):
    deg_o = da0[:, :1] + da1[:, :1]
    deg_i = db0[:, :1] + db1[:, :1]
    no = jnp.where(deg_o > 0, lax.rsqrt(deg_o), 0.0)
    ni = jnp.where(deg_i > 0, lax.rsqrt(deg_i), 0.0)
    s = sn[...]
    nout[...] = jnp.broadcast_to(no, (_BN, _D))
    ninsn[...] = jnp.broadcast_to(ni * s, (_BN, _D))
    snb[...] = jnp.broadcast_to(s, (_BN, _D))


_prep = pl.pallas_call(
    _prep_body,
    grid=(_G,),
    in_specs=[_row_spec()] * 4 + [pl.BlockSpec((_BN, 1), lambda i: (i, 0))],
    out_specs=[_row_spec()] * 3,
    out_shape=[jax.ShapeDtypeStruct((_N, _D), _f32)] * 3,
)


def _embed_body(h, we, be, nout, w1, x, t):
    xv = jnp.dot(h[...], we[...], preferred_element_type=_f32) + be[...]
    x[...] = xv
    t[...] = jnp.dot(xv * nout[...], w1[...], preferred_element_type=_f32)


_embed = pl.pallas_call(
    _embed_body,
    grid=(_G,),
    in_specs=[_row_spec(), _full_spec((_D, _D)), _full_spec((1, _D)),
              _row_spec(), _full_spec((_D, _D))],
    out_specs=[_row_spec(), _row_spec()],
    out_shape=[jax.ShapeDtypeStruct((_N, _D), _f32)] * 2,
)


def _post_body(p0, p1, ninsn, snb, b, u, ssum, ssq):
    uv = (p0[...] + p1[...]) * ninsn[...] + b[...] * snb[...]
    u[...] = uv

    @pl.when(pl.program_id(0) == 0)
    def _():
        ssum[...] = jnp.zeros((1, _D), _f32)
        ssq[...] = jnp.zeros((1, _D), _f32)

    ssum[...] += jnp.sum(uv, axis=0, keepdims=True)
    ssq[...] += jnp.sum(uv * uv, axis=0, keepdims=True)


_post = pl.pallas_call(
    _post_body,
    grid=(_G,),
    in_specs=[_row_spec(), _row_spec(), _row_spec(), _row_spec(),
              _full_spec((1, _D))],
    out_specs=[_row_spec(), _full_spec((1, _D)), _full_spec((1, _D))],
    out_shape=[jax.ShapeDtypeStruct((_N, _D), _f32),
               jax.ShapeDtypeStruct((1, _D), _f32),
               jax.ShapeDtypeStruct((1, _D), _f32)],
)


def _bn_x(u, ssum, ssq, g, bt, xp):
    mean = ssum[...] * (1.0 / _N)
    var = ssq[...] * (1.0 / _N) - mean * mean
    rstd = lax.rsqrt(var + 1e-5)
    return jax.nn.relu((u[...] - mean) * rstd * g[...] + bt[...]) + xp[...]


def _bnnext_body(u, ssum, ssq, g, bt, xp, nout, wn, x, t):
    xv = _bn_x(u, ssum, ssq, g, bt, xp)
    x[...] = xv
    t[...] = jnp.dot(xv * nout[...], wn[...], preferred_element_type=_f32)


_bnnext = pl.pallas_call(
    _bnnext_body,
    grid=(_G,),
    in_specs=[_row_spec(), _full_spec((1, _D)), _full_spec((1, _D)),
              _full_spec((1, _D)), _full_spec((1, _D)), _row_spec(),
              _row_spec(), _full_spec((_D, _D))],
    out_specs=[_row_spec(), _row_spec()],
    out_shape=[jax.ShapeDtypeStruct((_N, _D), _f32)] * 2,
)


def _bnlast_body(u, ssum, ssq, g, bt, xp, xsum):
    xv = _bn_x(u, ssum, ssq, g, bt, xp)

    @pl.when(pl.program_id(0) == 0)
    def _():
        xsum[...] = jnp.zeros((1, _D), _f32)

    xsum[...] += jnp.sum(xv, axis=0, keepdims=True)


_bnlast = pl.pallas_call(
    _bnlast_body,
    grid=(_G,),
    in_specs=[_row_spec(), _full_spec((1, _D)), _full_spec((1, _D)),
              _full_spec((1, _D)), _full_spec((1, _D)), _row_spec()],
    out_specs=[_full_spec((1, _D))],
    out_shape=[jax.ShapeDtypeStruct((1, _D), _f32)],
)


def _head_body(xsum, w1, b1, w2, b2, o):
    hg = xsum[...] * (1.0 / _N)
    z = jax.nn.relu(jnp.dot(hg, w1[...], preferred_element_type=_f32) + b1[...])
    o[...] = jnp.dot(z, w2[...], preferred_element_type=_f32) + b2[...]


_head = pl.pallas_call(
    _head_body,
    in_specs=[pl.BlockSpec((1, _D), lambda: (0, 0)),
              pl.BlockSpec((_D, _D), lambda: (0, 0)),
              pl.BlockSpec((1, _D), lambda: (0, 0)),
              pl.BlockSpec((_D, 10), lambda: (0, 0)),
              pl.BlockSpec((1, 10), lambda: (0, 0))],
    out_specs=pl.BlockSpec((1, 10), lambda: (0, 0)),
    out_shape=jax.ShapeDtypeStruct((1, 10), _f32),
)


def kernel(h, edge_index, e, snorm_n, snorm_e, W_emb, b_emb, Ws, bs,
           gammas, betas, W_mlp1, b_mlp1, W_mlp2, b_mlp2):
    del e, snorm_e
    src3 = edge_index[0].reshape(_NW, _NCHUNK, _CH)
    dst3 = edge_index[1].reshape(_NW, _NCHUNK, _CH)
    src2 = edge_index[0].reshape(_NW, _EPW)
    onesd = jnp.ones((_CH, _D), _f32)
    znd = jnp.zeros((_NP, _D), _f32)

    _sc_degrees, _sc_aggregate = _sc_kernels()
    da, db = _sc_degrees(src3, dst3, onesd, znd)
    nout, ninsn, snb = _prep(da[:_N], da[_NP:_NP + _N], db[:_N], db[_NP:_NP + _N], snorm_n)

    x, t = _embed(h, W_emb, b_emb.reshape(1, _D), nout, Ws[0])
    n_layers = len(Ws)
    for i in range(n_layers):
        p = _sc_aggregate(t, src2, dst3, znd)
        u, ssum, ssq = _post(p[:_N], p[_NP:_NP + _N], ninsn, snb, bs[i].reshape(1, _D))
        g2 = gammas[i].reshape(1, _D)
        bt2 = betas[i].reshape(1, _D)
        if i < n_layers - 1:
            x, t = _bnnext(u, ssum, ssq, g2, bt2, x, nout, Ws[i + 1])
        else:
            (xsum,) = _bnlast(u, ssum, ssq, g2, bt2, x)

    return _head(xsum, W_mlp1, b_mlp1.reshape(1, _D),
                 W_mlp2, b_mlp2.reshape(1, 10))


# speedup vs baseline: 8.2770x; 1.0162x over previous
import functools

import jax
import jax.numpy as jnp
from jax import lax
from jax.experimental import pallas as pl
from jax.experimental.pallas import tpu as pltpu
from jax.experimental.pallas import tpu_sc as plsc

_N = 10000
_E = 320000
_D = 128
_NC = 2
_NS = 16
_NW = _NC * _NS
_EPW = _E // _NW
_CH = 80
_NCHUNK = _EPW // _CH
_NP = 10240
_RPT = _NP // _NS
_BN = 2000
_G = _N // _BN

_f32 = jnp.float32


def _sc_degrees_body(src_hbm, dst_hbm, onesa_hbm, onesb_hbm, z_hbm, out_hbm,
                     eidx, onesa_v, onesb_v, tab):
    c = lax.axis_index("c")
    s = lax.axis_index("s")
    wid = s * _NC + c
    my = pl.ds(s * _RPT, _RPT)
    pltpu.sync_copy(src_hbm.at[wid], eidx)
    pltpu.sync_copy(onesa_hbm, onesa_v)
    pltpu.sync_copy(onesb_hbm, onesb_v)
    pltpu.sync_copy(z_hbm.at[my], tab.at[my])
    plsc.subcore_barrier()

    @pl.loop(0, _NCHUNK)
    def _(i):
        pltpu.sync_copy(onesa_v, tab.at[eidx.at[i]], add=True)

    pltpu.sync_copy(dst_hbm.at[wid], eidx)

    @pl.loop(0, _NCHUNK)
    def _(i):
        pltpu.sync_copy(onesb_v, tab.at[eidx.at[i]], add=True)

    plsc.subcore_barrier()
    pltpu.sync_copy(tab.at[my], out_hbm.at[pl.ds(c * _NP + s * _RPT, _RPT)])


def _sc_aggregate_body(t_hbm, src_hbm, dst_hbm, znd_hbm, out_hbm,
                       sidx, didx, r0, r1, acc, sg0, sg1):
    c = lax.axis_index("c")
    s = lax.axis_index("s")
    wid = s * _NC + c
    my = pl.ds(s * _RPT, _RPT)
    pltpu.sync_copy(src_hbm.at[wid], sidx)
    pltpu.sync_copy(dst_hbm.at[wid], didx)
    pltpu.sync_copy(znd_hbm.at[my], acc.at[my])
    plsc.subcore_barrier()

    def gstart(i, buf, sem):
        pltpu.async_copy(t_hbm.at[sidx.at[pl.ds(i * _CH, _CH)]], buf, sem)

    def gwait(buf, sem):
        pltpu.make_async_copy(t_hbm.at[pl.ds(0, _CH)], buf, sem).wait()

    def sadd(i, buf):
        pltpu.sync_copy(buf, acc.at[didx.at[i]], add=True)

    gstart(0, r0, sg0)
    gstart(1, r1, sg1)

    @pl.loop(0, (_NCHUNK - 3) // 2)
    def _(j):
        i = 2 * j
        gwait(r0, sg0)
        sadd(i, r0)
        gstart(i + 2, r0, sg0)
        gwait(r1, sg1)
        sadd(i + 1, r1)
        gstart(i + 3, r1, sg1)

    gwait(r0, sg0)
    sadd(_NCHUNK - 3, r0)
    gstart(_NCHUNK - 1, r0, sg0)
    gwait(r1, sg1)
    sadd(_NCHUNK - 2, r1)
    gwait(r0, sg0)
    sadd(_NCHUNK - 1, r0)

    plsc.subcore_barrier()
    pltpu.sync_copy(acc.at[my], out_hbm.at[pl.ds(c * _NP + s * _RPT, _RPT)])


@functools.cache
def _sc_kernels():
    mesh = plsc.VectorSubcoreMesh(core_axis_name="c", subcore_axis_name="s",
                                  num_cores=_NC, num_subcores=_NS)
    degrees = pl.kernel(
        _sc_degrees_body,
        mesh=mesh,
        out_type=jax.ShapeDtypeStruct((_NC * _NP, _D), _f32),
        scratch_types=[pltpu.VMEM((_NCHUNK, _CH), jnp.int32),
                       pltpu.VMEM((_CH, _D), _f32),
                       pltpu.VMEM((_CH, _D), _f32),
                       pltpu.VMEM_SHARED((_NP, _D), _f32)],
    )
    aggregate = pl.kernel(
        _sc_aggregate_body,
        mesh=mesh,
        out_type=jax.ShapeDtypeStruct((_NC * _NP, _D), _f32),
        scratch_types=[pltpu.VMEM((_EPW,), jnp.int32),
                       pltpu.VMEM((_NCHUNK, _CH), jnp.int32),
                       pltpu.VMEM((_CH, _D), _f32),
                       pltpu.VMEM((_CH, _D), _f32),
                       pltpu.VMEM_SHARED((_NP, _D), _f32),
                       pltpu.SemaphoreType.DMA,
                       pltpu.SemaphoreType.DMA],
    )
    return degrees, aggregate


def _row_spec():
    return pl.BlockSpec((_BN, _D), lambda i: (i, 0))


def _full_spec(shape):
    return pl.BlockSpec(shape, lambda i: tuple(0 for _ in shape))


def _prep_body(d0, d1, sn, nout, ninsn, snb):
    deg_o = d0[:, :1] + d1[:, :1]
    deg_i = d0[:, 64:65] + d1[:, 64:65]
    no = jnp.where(deg_o > 0, lax.rsqrt(deg_o), 0.0)
    ni = jnp.where(deg_i > 0, lax.rsqrt(deg_i), 0.0)
    s = sn[...]
    nout[...] = jnp.broadcast_to(no, (_BN, _D))
    ninsn[...] = jnp.broadcast_to(ni * s, (_BN, _D))
    snb[...] = jnp.broadcast_to(s, (_BN, _D))


_prep = pl.pallas_call(
    _prep_body,
    grid=(_G,),
    in_specs=[_row_spec()] * 2 + [pl.BlockSpec((_BN, 1), lambda i: (i, 0))],
    out_specs=[_row_spec()] * 3,
    out_shape=[jax.ShapeDtypeStruct((_N, _D), _f32)] * 3,
)


def _embed_body(h, we, be, nout, w1, x, t):
    xv = jnp.dot(h[...], we[...], preferred_element_type=_f32) + be[...]
    x[...] = xv
    t[...] = jnp.dot(xv * nout[...], w1[...], preferred_element_type=_f32)


_embed = pl.pallas_call(
    _embed_body,
    grid=(_G,),
    in_specs=[_row_spec(), _full_spec((_D, _D)), _full_spec((1, _D)),
              _row_spec(), _full_spec((_D, _D))],
    out_specs=[_row_spec(), _row_spec()],
    out_shape=[jax.ShapeDtypeStruct((_N, _D), _f32)] * 2,
)


def _post_body(p0, p1, ninsn, snb, b, u, ssum, ssq):
    uv = (p0[...] + p1[...]) * ninsn[...] + b[...] * snb[...]
    u[...] = uv

    @pl.when(pl.program_id(0) == 0)
    def _():
        ssum[...] = jnp.zeros((1, _D), _f32)
        ssq[...] = jnp.zeros((1, _D), _f32)

    ssum[...] += jnp.sum(uv, axis=0, keepdims=True)
    ssq[...] += jnp.sum(uv * uv, axis=0, keepdims=True)


_post = pl.pallas_call(
    _post_body,
    grid=(_G,),
    in_specs=[_row_spec(), _row_spec(), _row_spec(), _row_spec(),
              _full_spec((1, _D))],
    out_specs=[_row_spec(), _full_spec((1, _D)), _full_spec((1, _D))],
    out_shape=[jax.ShapeDtypeStruct((_N, _D), _f32),
               jax.ShapeDtypeStruct((1, _D), _f32),
               jax.ShapeDtypeStruct((1, _D), _f32)],
)


def _bn_x(u, ssum, ssq, g, bt, xp):
    mean = ssum[...] * (1.0 / _N)
    var = ssq[...] * (1.0 / _N) - mean * mean
    rstd = lax.rsqrt(var + 1e-5)
    return jax.nn.relu((u[...] - mean) * rstd * g[...] + bt[...]) + xp[...]


def _bnnext_body(u, ssum, ssq, g, bt, xp, nout, wn, x, t):
    xv = _bn_x(u, ssum, ssq, g, bt, xp)
    x[...] = xv
    t[...] = jnp.dot(xv * nout[...], wn[...], preferred_element_type=_f32)


_bnnext = pl.pallas_call(
    _bnnext_body,
    grid=(_G,),
    in_specs=[_row_spec(), _full_spec((1, _D)), _full_spec((1, _D)),
              _full_spec((1, _D)), _full_spec((1, _D)), _row_spec(),
              _row_spec(), _full_spec((_D, _D))],
    out_specs=[_row_spec(), _row_spec()],
    out_shape=[jax.ShapeDtypeStruct((_N, _D), _f32)] * 2,
)


def _bnlast_body(u, ssum, ssq, g, bt, xp, xsum):
    xv = _bn_x(u, ssum, ssq, g, bt, xp)

    @pl.when(pl.program_id(0) == 0)
    def _():
        xsum[...] = jnp.zeros((1, _D), _f32)

    xsum[...] += jnp.sum(xv, axis=0, keepdims=True)


_bnlast = pl.pallas_call(
    _bnlast_body,
    grid=(_G,),
    in_specs=[_row_spec(), _full_spec((1, _D)), _full_spec((1, _D)),
              _full_spec((1, _D)), _full_spec((1, _D)), _row_spec()],
    out_specs=[_full_spec((1, _D))],
    out_shape=[jax.ShapeDtypeStruct((1, _D), _f32)],
)


def _head_body(xsum, w1, b1, w2, b2, o):
    hg = xsum[...] * (1.0 / _N)
    z = jax.nn.relu(jnp.dot(hg, w1[...], preferred_element_type=_f32) + b1[...])
    o[...] = jnp.dot(z, w2[...], preferred_element_type=_f32) + b2[...]


_head = pl.pallas_call(
    _head_body,
    in_specs=[pl.BlockSpec((1, _D), lambda: (0, 0)),
              pl.BlockSpec((_D, _D), lambda: (0, 0)),
              pl.BlockSpec((1, _D), lambda: (0, 0)),
              pl.BlockSpec((_D, 10), lambda: (0, 0)),
              pl.BlockSpec((1, 10), lambda: (0, 0))],
    out_specs=pl.BlockSpec((1, 10), lambda: (0, 0)),
    out_shape=jax.ShapeDtypeStruct((1, 10), _f32),
)


def kernel(h, edge_index, e, snorm_n, snorm_e, W_emb, b_emb, Ws, bs,
           gammas, betas, W_mlp1, b_mlp1, W_mlp2, b_mlp2):
    del e, snorm_e
    src3 = edge_index[0].reshape(_NW, _NCHUNK, _CH)
    dst3 = edge_index[1].reshape(_NW, _NCHUNK, _CH)
    src2 = edge_index[0].reshape(_NW, _EPW)
    lane = jnp.arange(_D)
    onesa = jnp.broadcast_to((lane == 0).astype(_f32), (_CH, _D))
    onesb = jnp.broadcast_to((lane == 64).astype(_f32), (_CH, _D))
    znd = jnp.zeros((_NP, _D), _f32)

    _sc_degrees, _sc_aggregate = _sc_kernels()
    dd = _sc_degrees(src3, dst3, onesa, onesb, znd)
    nout, ninsn, snb = _prep(dd[:_N], dd[_NP:_NP + _N], snorm_n)

    x, t = _embed(h, W_emb, b_emb.reshape(1, _D), nout, Ws[0])
    n_layers = len(Ws)
    for i in range(n_layers):
        p = _sc_aggregate(t, src2, dst3, znd)
        u, ssum, ssq = _post(p[:_N], p[_NP:_NP + _N], ninsn, snb, bs[i].reshape(1, _D))
        g2 = gammas[i].reshape(1, _D)
        bt2 = betas[i].reshape(1, _D)
        if i < n_layers - 1:
            x, t = _bnnext(u, ssum, ssq, g2, bt2, x, nout, Ws[i + 1])
        else:
            (xsum,) = _bnlast(u, ssum, ssq, g2, bt2, x)

    return _head(xsum, W_mlp1, b_mlp1.reshape(1, _D),
                 W_mlp2, b_mlp2.reshape(1, 10))


# speedup vs baseline: 8.5069x; 1.0278x over previous
import functools

import jax
import jax.numpy as jnp
from jax import lax
from jax.experimental import pallas as pl
from jax.experimental.pallas import tpu as pltpu
from jax.experimental.pallas import tpu_sc as plsc

_N = 10000
_E = 320000
_D = 128
_NC = 2
_NS = 16
_NW = _NC * _NS
_EPW = _E // _NW
_CH = 80
_NCHUNK = _EPW // _CH
_NP = 10240
_RPT = _NP // _NS
_BN = 2000
_G = _N // _BN

_f32 = jnp.float32


def _sc_degrees_body(src_hbm, dst_hbm, onesa_hbm, onesb_hbm, z_hbm, out_hbm,
                     eidx, onesa_v, onesb_v, tab):
    c = lax.axis_index("c")
    s = lax.axis_index("s")
    wid = s * _NC + c
    my = pl.ds(s * _RPT, _RPT)
    pltpu.sync_copy(src_hbm.at[wid], eidx)
    pltpu.sync_copy(onesa_hbm, onesa_v)
    pltpu.sync_copy(onesb_hbm, onesb_v)
    pltpu.sync_copy(z_hbm.at[my], tab.at[my])
    plsc.subcore_barrier()

    @pl.loop(0, _NCHUNK)
    def _(i):
        pltpu.sync_copy(onesa_v, tab.at[eidx.at[i]], add=True)

    pltpu.sync_copy(dst_hbm.at[wid], eidx)

    @pl.loop(0, _NCHUNK)
    def _(i):
        pltpu.sync_copy(onesb_v, tab.at[eidx.at[i]], add=True)

    plsc.subcore_barrier()
    pltpu.sync_copy(tab.at[my], out_hbm.at[pl.ds(c * _NP + s * _RPT, _RPT)])


def _sc_aggregate_body(t_hbm, src_hbm, dst_hbm, znd_hbm, out_hbm,
                       sidx, didx, r0, r1, acc, sg0, sg1):
    c = lax.axis_index("c")
    s = lax.axis_index("s")
    wid = s * _NC + c
    my = pl.ds(s * _RPT, _RPT)
    pltpu.sync_copy(src_hbm.at[wid], sidx)
    pltpu.sync_copy(dst_hbm.at[wid], didx)
    pltpu.sync_copy(znd_hbm.at[my], acc.at[my])
    plsc.subcore_barrier()

    def gstart(i, buf, sem):
        pltpu.async_copy(t_hbm.at[sidx.at[pl.ds(i * _CH, _CH)]], buf, sem)

    def gwait(buf, sem):
        pltpu.make_async_copy(t_hbm.at[pl.ds(0, _CH)], buf, sem).wait()

    def sadd(i, buf):
        pltpu.sync_copy(buf, acc.at[didx.at[i]], add=True)

    gstart(0, r0, sg0)
    gstart(1, r1, sg1)

    @pl.loop(0, (_NCHUNK - 3) // 2)
    def _(j):
        i = 2 * j
        gwait(r0, sg0)
        sadd(i, r0)
        gstart(i + 2, r0, sg0)
        gwait(r1, sg1)
        sadd(i + 1, r1)
        gstart(i + 3, r1, sg1)

    gwait(r0, sg0)
    sadd(_NCHUNK - 3, r0)
    gstart(_NCHUNK - 1, r0, sg0)
    gwait(r1, sg1)
    sadd(_NCHUNK - 2, r1)
    gwait(r0, sg0)
    sadd(_NCHUNK - 1, r0)

    plsc.subcore_barrier()
    pltpu.sync_copy(acc.at[my], out_hbm.at[pl.ds(c * _NP + s * _RPT, _RPT)])


@functools.cache
def _sc_kernels():
    mesh = plsc.VectorSubcoreMesh(core_axis_name="c", subcore_axis_name="s",
                                  num_cores=_NC, num_subcores=_NS)
    degrees = pl.kernel(
        _sc_degrees_body,
        mesh=mesh,
        out_type=jax.ShapeDtypeStruct((_NC * _NP, _D), _f32),
        scratch_types=[pltpu.VMEM((_NCHUNK, _CH), jnp.int32),
                       pltpu.VMEM((_CH, _D), _f32),
                       pltpu.VMEM((_CH, _D), _f32),
                       pltpu.VMEM_SHARED((_NP, _D), _f32)],
    )
    aggregate = pl.kernel(
        _sc_aggregate_body,
        mesh=mesh,
        out_type=jax.ShapeDtypeStruct((_NC * _NP, _D), _f32),
        scratch_types=[pltpu.VMEM((_EPW,), jnp.int32),
                       pltpu.VMEM((_NCHUNK, _CH), jnp.int32),
                       pltpu.VMEM((_CH, _D), _f32),
                       pltpu.VMEM((_CH, _D), _f32),
                       pltpu.VMEM_SHARED((_NP, _D), _f32),
                       pltpu.SemaphoreType.DMA,
                       pltpu.SemaphoreType.DMA],
    )
    return degrees, aggregate


def _row_spec():
    return pl.BlockSpec((_BN, _D), lambda i: (i, 0))


def _full_spec(shape):
    return pl.BlockSpec(shape, lambda i: tuple(0 for _ in shape))


def _embed_body(h, we, be, d0, d1, sn, w1, x, t, nout, ninsn, snb):
    deg_o = d0[:, :1] + d1[:, :1]
    deg_i = d0[:, 64:65] + d1[:, 64:65]
    no = jnp.where(deg_o > 0, lax.rsqrt(deg_o), 0.0)
    ni = jnp.where(deg_i > 0, lax.rsqrt(deg_i), 0.0)
    s = sn[...]
    nob = jnp.broadcast_to(no, (_BN, _D))
    nout[...] = nob
    ninsn[...] = jnp.broadcast_to(ni * s, (_BN, _D))
    snb[...] = jnp.broadcast_to(s, (_BN, _D))
    xv = jnp.dot(h[...], we[...], preferred_element_type=_f32) + be[...]
    x[...] = xv
    t[...] = jnp.dot(xv * nob, w1[...], preferred_element_type=_f32)


_embed = pl.pallas_call(
    _embed_body,
    grid=(_G,),
    in_specs=[_row_spec(), _full_spec((_D, _D)), _full_spec((1, _D)),
              _row_spec(), _row_spec(), pl.BlockSpec((_BN, 1), lambda i: (i, 0)),
              _full_spec((_D, _D))],
    out_specs=[_row_spec()] * 5,
    out_shape=[jax.ShapeDtypeStruct((_N, _D), _f32)] * 5,
)


def _stats_phase(p0, p1, ninsn, snb, b, u_s, st_s, i):
    uv = (p0[...] + p1[...]) * ninsn[...] + b[...] * snb[...]
    u_s[pl.ds(i * _BN, _BN), :] = uv

    @pl.when(i == 0)
    def _():
        st_s[...] = jnp.zeros((2, _D), _f32)

    st_s[0:1, :] += jnp.sum(uv, axis=0, keepdims=True)
    st_s[1:2, :] += jnp.sum(uv * uv, axis=0, keepdims=True)


def _apply_phase(g, bt, xp, u_s, st_s, i):
    mean = st_s[0:1, :] * (1.0 / _N)
    var = st_s[1:2, :] * (1.0 / _N) - mean * mean
    rstd = lax.rsqrt(var + 1e-5)
    uv = u_s[pl.ds(i * _BN, _BN), :]
    return jax.nn.relu((uv - mean) * rstd * g[...] + bt[...]) + xp[...]


def _layer_mid_body(p0, p1, ninsn, snb, b, g, bt, xp, nout, wn, x, t, u_s, st_s):
    ph = pl.program_id(0)
    i = pl.program_id(1)

    @pl.when(ph == 0)
    def _():
        _stats_phase(p0, p1, ninsn, snb, b, u_s, st_s, i)

    @pl.when(ph == 1)
    def _():
        xv = _apply_phase(g, bt, xp, u_s, st_s, i)
        x[...] = xv
        t[...] = jnp.dot(xv * nout[...], wn[...], preferred_element_type=_f32)


def _p0_spec():
    return pl.BlockSpec((_BN, _D), lambda ph, i: ((1 - ph) * i, 0))


def _p1_spec():
    return pl.BlockSpec((_BN, _D), lambda ph, i: (ph * i, 0))


def _c_spec(shape):
    return pl.BlockSpec(shape, lambda ph, i: tuple(0 for _ in shape))


_layer_mid = pl.pallas_call(
    _layer_mid_body,
    grid=(2, _G),
    in_specs=[_p0_spec(), _p0_spec(), _p0_spec(), _p0_spec(),
              _c_spec((1, _D)), _c_spec((1, _D)), _c_spec((1, _D)),
              _p1_spec(), _p1_spec(), _c_spec((_D, _D))],
    out_specs=[_p1_spec(), _p1_spec()],
    out_shape=[jax.ShapeDtypeStruct((_N, _D), _f32)] * 2,
    scratch_shapes=[pltpu.VMEM((_N, _D), _f32), pltpu.VMEM((2, _D), _f32)],
)


def _layer_last_body(p0, p1, ninsn, snb, b, g, bt, xp, w1, b1, w2, b2, o,
                     u_s, st_s, xs_s):
    ph = pl.program_id(0)
    i = pl.program_id(1)

    @pl.when(ph == 0)
    def _():
        _stats_phase(p0, p1, ninsn, snb, b, u_s, st_s, i)

    @pl.when(ph == 1)
    def _():
        xv = _apply_phase(g, bt, xp, u_s, st_s, i)

        @pl.when(i == 0)
        def _():
            xs_s[...] = jnp.zeros((1, _D), _f32)

        xs_s[...] += jnp.sum(xv, axis=0, keepdims=True)

        @pl.when(i == _G - 1)
        def _():
            hg = xs_s[...] * (1.0 / _N)
            z = jax.nn.relu(jnp.dot(hg, w1[...], preferred_element_type=_f32)
                            + b1[...])
            o[...] = jnp.dot(z, w2[...], preferred_element_type=_f32) + b2[...]


_layer_last = pl.pallas_call(
    _layer_last_body,
    grid=(2, _G),
    in_specs=[_p0_spec(), _p0_spec(), _p0_spec(), _p0_spec(),
              _c_spec((1, _D)), _c_spec((1, _D)), _c_spec((1, _D)),
              _p1_spec(), _c_spec((_D, _D)), _c_spec((1, _D)),
              _c_spec((_D, 10)), _c_spec((1, 10))],
    out_specs=[_c_spec((1, 10))],
    out_shape=[jax.ShapeDtypeStruct((1, 10), _f32)],
    scratch_shapes=[pltpu.VMEM((_N, _D), _f32), pltpu.VMEM((2, _D), _f32),
                    pltpu.VMEM((1, _D), _f32)],
)


def kernel(h, edge_index, e, snorm_n, snorm_e, W_emb, b_emb, Ws, bs,
           gammas, betas, W_mlp1, b_mlp1, W_mlp2, b_mlp2):
    del e, snorm_e
    src3 = edge_index[0].reshape(_NW, _NCHUNK, _CH)
    dst3 = edge_index[1].reshape(_NW, _NCHUNK, _CH)
    src2 = edge_index[0].reshape(_NW, _EPW)
    lane = jnp.arange(_D)
    onesa = jnp.broadcast_to((lane == 0).astype(_f32), (_CH, _D))
    onesb = jnp.broadcast_to((lane == 64).astype(_f32), (_CH, _D))
    znd = jnp.zeros((_NP, _D), _f32)

    _sc_degrees, _sc_aggregate = _sc_kernels()
    dd = _sc_degrees(src3, dst3, onesa, onesb, znd)
    x, t, nout, ninsn, snb = _embed(h, W_emb, b_emb.reshape(1, _D),
                                    dd[:_N], dd[_NP:_NP + _N], snorm_n, Ws[0])
    n_layers = len(Ws)
    for i in range(n_layers):
        p = _sc_aggregate(t, src2, dst3, znd)
        b2 = bs[i].reshape(1, _D)
        g2 = gammas[i].reshape(1, _D)
        bt2 = betas[i].reshape(1, _D)
        if i < n_layers - 1:
            x, t = _layer_mid(p[:_N], p[_NP:_NP + _N], ninsn, snb, b2, g2,
                              bt2, x, nout, Ws[i + 1])
        else:
            (out,) = _layer_last(p[:_N], p[_NP:_NP + _N], ninsn, snb, b2, g2,
                                 bt2, x, W_mlp1, b_mlp1.reshape(1, _D),
                                 W_mlp2, b_mlp2.reshape(1, 10))
    return out


# speedup vs baseline: 9.9496x; 1.1696x over previous
import functools

import jax
import jax.numpy as jnp
from jax import lax
from jax.experimental import pallas as pl
from jax.experimental.pallas import tpu as pltpu
from jax.experimental.pallas import tpu_sc as plsc

_N = 10000
_E = 320000
_D = 128
_NC = 2
_NS = 16
_NW = _NC * _NS
_EPW = _E // _NW
_CH = 80
_NCHUNK = _EPW // _CH
_NP = 10240
_RPT = _NP // _NS
_BN = 2000
_G = _N // _BN

_f32 = jnp.float32


def _sc_degrees_body(src_hbm, dst_hbm, znd_hbm, i80_hbm, out_hbm,
                     e1d, ta, tb, i80, tsha, tshb):
    c = lax.axis_index("c")
    s = lax.axis_index("s")
    wid = s * _NC + c
    zv = jnp.zeros((16,), _f32)
    ones = jnp.full((16,), 1.0, _f32)
    pltpu.sync_copy(i80_hbm, i80)

    @pl.when(s == 0)
    def _():
        pltpu.sync_copy(znd_hbm.at[pl.ds(0, 80)], tsha)
        pltpu.sync_copy(znd_hbm.at[pl.ds(0, 80)], tshb)

    @pl.loop(0, 80)
    def _(r):
        @pl.loop(0, 8)
        def _(k):
            ta[r, pl.ds(k * 16, 16)] = zv
            tb[r, pl.ds(k * 16, 16)] = zv

    pltpu.sync_copy(src_hbm.at[wid], e1d)

    @pl.loop(0, _EPW // 16)
    def _(j):
        v = e1d[pl.ds(j * 16, 16)]
        plsc.addupdate_scatter(ta, [jnp.right_shift(v, 7),
                                    jnp.bitwise_and(v, 127)], ones)

    pltpu.sync_copy(dst_hbm.at[wid], e1d)

    @pl.loop(0, _EPW // 16)
    def _(j):
        v = e1d[pl.ds(j * 16, 16)]
        plsc.addupdate_scatter(tb, [jnp.right_shift(v, 7),
                                    jnp.bitwise_and(v, 127)], ones)

    plsc.subcore_barrier()
    pltpu.sync_copy(ta, tsha.at[i80.at[0]], add=True)
    pltpu.sync_copy(tb, tshb.at[i80.at[0]], add=True)
    plsc.subcore_barrier()

    @pl.when(s == 0)
    def _():
        pltpu.sync_copy(tsha, out_hbm.at[pl.ds(c * 160, 80)])

    @pl.when(s == 1)
    def _():
        pltpu.sync_copy(tshb, out_hbm.at[pl.ds(c * 160 + 80, 80)])


def _sc_aggregate_body(t_hbm, src_hbm, dst_hbm, znd_hbm, out_hbm,
                       sidx, didx, r0, r1, acc, sg0, sg1):
    c = lax.axis_index("c")
    s = lax.axis_index("s")
    wid = s * _NC + c
    my = pl.ds(s * _RPT, _RPT)
    pltpu.sync_copy(src_hbm.at[wid], sidx)
    pltpu.sync_copy(dst_hbm.at[wid], didx)
    pltpu.sync_copy(znd_hbm.at[my], acc.at[my])
    plsc.subcore_barrier()

    def gstart(i, buf, sem):
        pltpu.async_copy(t_hbm.at[sidx.at[pl.ds(i * _CH, _CH)]], buf, sem)

    def gwait(buf, sem):
        pltpu.make_async_copy(t_hbm.at[pl.ds(0, _CH)], buf, sem).wait()

    def sadd(i, buf):
        pltpu.sync_copy(buf, acc.at[didx.at[i]], add=True)

    gstart(0, r0, sg0)
    gstart(1, r1, sg1)

    @pl.loop(0, (_NCHUNK - 3) // 2)
    def _(j):
        i = 2 * j
        gwait(r0, sg0)
        sadd(i, r0)
        gstart(i + 2, r0, sg0)
        gwait(r1, sg1)
        sadd(i + 1, r1)
        gstart(i + 3, r1, sg1)

    gwait(r0, sg0)
    sadd(_NCHUNK - 3, r0)
    gstart(_NCHUNK - 1, r0, sg0)
    gwait(r1, sg1)
    sadd(_NCHUNK - 2, r1)
    gwait(r0, sg0)
    sadd(_NCHUNK - 1, r0)

    plsc.subcore_barrier()
    pltpu.sync_copy(acc.at[my], out_hbm.at[pl.ds(c * _NP + s * _RPT, _RPT)])


@functools.cache
def _sc_kernels():
    import dataclasses
    mesh = plsc.VectorSubcoreMesh(core_axis_name="c", subcore_axis_name="s",
                                  num_cores=_NC, num_subcores=_NS)
    cp = pltpu.CompilerParams()
    if "needs_layout_passes" in pltpu.CompilerParams.__dataclass_fields__:
        cp = dataclasses.replace(cp, needs_layout_passes=False)
    degrees = pl.kernel(
        _sc_degrees_body,
        mesh=mesh,
        out_type=jax.ShapeDtypeStruct((_NC * 160, _D), _f32),
        scratch_types=[pltpu.VMEM((_EPW,), jnp.int32),
                       pltpu.VMEM((80, _D), _f32),
                       pltpu.VMEM((80, _D), _f32),
                       pltpu.VMEM((1, 80), jnp.int32),
                       pltpu.VMEM_SHARED((80, _D), _f32),
                       pltpu.VMEM_SHARED((80, _D), _f32)],
        compiler_params=cp,
    )
    aggregate = pl.kernel(
        _sc_aggregate_body,
        mesh=mesh,
        out_type=jax.ShapeDtypeStruct((_NC * _NP, _D), _f32),
        scratch_types=[pltpu.VMEM((_EPW,), jnp.int32),
                       pltpu.VMEM((_NCHUNK, _CH), jnp.int32),
                       pltpu.VMEM((_CH, _D), _f32),
                       pltpu.VMEM((_CH, _D), _f32),
                       pltpu.VMEM_SHARED((_NP, _D), _f32),
                       pltpu.SemaphoreType.DMA,
                       pltpu.SemaphoreType.DMA],
    )
    return degrees, aggregate


def _row_spec():
    return pl.BlockSpec((_BN, _D), lambda i: (i, 0))


def _full_spec(shape):
    return pl.BlockSpec(shape, lambda i: tuple(0 for _ in shape))


def _embed_body(h, we, be, d0, d1, sn, w1, x, t, nout, ninsn, snb):
    deg_o = d0[...]
    deg_i = d1[...]
    no = jnp.where(deg_o > 0, lax.rsqrt(deg_o), 0.0)
    ni = jnp.where(deg_i > 0, lax.rsqrt(deg_i), 0.0)
    s = sn[...]
    nob = jnp.broadcast_to(no, (_BN, _D))
    nout[...] = nob
    ninsn[...] = jnp.broadcast_to(ni * s, (_BN, _D))
    snb[...] = jnp.broadcast_to(s, (_BN, _D))
    xv = jnp.dot(h[...], we[...], preferred_element_type=_f32) + be[...]
    x[...] = xv
    t[...] = jnp.dot(xv * nob, w1[...], preferred_element_type=_f32)


_embed = pl.pallas_call(
    _embed_body,
    grid=(_G,),
    in_specs=[_row_spec(), _full_spec((_D, _D)), _full_spec((1, _D)),
              pl.BlockSpec((_BN, 1), lambda i: (i, 0)),
              pl.BlockSpec((_BN, 1), lambda i: (i, 0)),
              pl.BlockSpec((_BN, 1), lambda i: (i, 0)),
              _full_spec((_D, _D))],
    out_specs=[_row_spec()] * 5,
    out_shape=[jax.ShapeDtypeStruct((_N, _D), _f32)] * 5,
)


def _stats_phase(p0, p1, ninsn, snb, b, u_s, st_s, i):
    uv = (p0[...] + p1[...]) * ninsn[...] + b[...] * snb[...]
    u_s[pl.ds(i * _BN, _BN), :] = uv

    @pl.when(i == 0)
    def _():
        st_s[...] = jnp.zeros((2, _D), _f32)

    st_s[0:1, :] += jnp.sum(uv, axis=0, keepdims=True)
    st_s[1:2, :] += jnp.sum(uv * uv, axis=0, keepdims=True)


def _apply_phase(g, bt, xp, u_s, st_s, i):
    mean = st_s[0:1, :] * (1.0 / _N)
    var = st_s[1:2, :] * (1.0 / _N) - mean * mean
    rstd = lax.rsqrt(var + 1e-5)
    uv = u_s[pl.ds(i * _BN, _BN), :]
    return jax.nn.relu((uv - mean) * rstd * g[...] + bt[...]) + xp[...]


def _layer_mid_body(p0, p1, ninsn, snb, b, g, bt, xp, nout, wn, x, t, u_s, st_s):
    ph = pl.program_id(0)
    i = pl.program_id(1)

    @pl.when(ph == 0)
    def _():
        _stats_phase(p0, p1, ninsn, snb, b, u_s, st_s, i)

    @pl.when(ph == 1)
    def _():
        xv = _apply_phase(g, bt, xp, u_s, st_s, i)
        x[...] = xv
        t[...] = jnp.dot(xv * nout[...], wn[...], preferred_element_type=_f32)


def _p0_spec():
    return pl.BlockSpec((_BN, _D), lambda ph, i: ((1 - ph) * i, 0))


def _p1_spec():
    return pl.BlockSpec((_BN, _D), lambda ph, i: (ph * i, 0))


def _c_spec(shape):
    return pl.BlockSpec(shape, lambda ph, i: tuple(0 for _ in shape))


_layer_mid = pl.pallas_call(
    _layer_mid_body,
    grid=(2, _G),
    in_specs=[_p0_spec(), _p0_spec(), _p0_spec(), _p0_spec(),
              _c_spec((1, _D)), _c_spec((1, _D)), _c_spec((1, _D)),
              _p1_spec(), _p1_spec(), _c_spec((_D, _D))],
    out_specs=[_p1_spec(), _p1_spec()],
    out_shape=[jax.ShapeDtypeStruct((_N, _D), _f32)] * 2,
    scratch_shapes=[pltpu.VMEM((_N, _D), _f32), pltpu.VMEM((2, _D), _f32)],
)


def _layer_last_body(p0, p1, ninsn, snb, b, g, bt, xp, w1, b1, w2, b2, o,
                     u_s, st_s, xs_s):
    ph = pl.program_id(0)
    i = pl.program_id(1)

    @pl.when(ph == 0)
    def _():
        _stats_phase(p0, p1, ninsn, snb, b, u_s, st_s, i)

    @pl.when(ph == 1)
    def _():
        xv = _apply_phase(g, bt, xp, u_s, st_s, i)

        @pl.when(i == 0)
        def _():
            xs_s[...] = jnp.zeros((1, _D), _f32)

        xs_s[...] += jnp.sum(xv, axis=0, keepdims=True)

        @pl.when(i == _G - 1)
        def _():
            hg = xs_s[...] * (1.0 / _N)
            z = jax.nn.relu(jnp.dot(hg, w1[...], preferred_element_type=_f32)
                            + b1[...])
            o[...] = jnp.dot(z, w2[...], preferred_element_type=_f32) + b2[...]


_layer_last = pl.pallas_call(
    _layer_last_body,
    grid=(2, _G),
    in_specs=[_p0_spec(), _p0_spec(), _p0_spec(), _p0_spec(),
              _c_spec((1, _D)), _c_spec((1, _D)), _c_spec((1, _D)),
              _p1_spec(), _c_spec((_D, _D)), _c_spec((1, _D)),
              _c_spec((_D, 10)), _c_spec((1, 10))],
    out_specs=[_c_spec((1, 10))],
    out_shape=[jax.ShapeDtypeStruct((1, 10), _f32)],
    scratch_shapes=[pltpu.VMEM((_N, _D), _f32), pltpu.VMEM((2, _D), _f32),
                    pltpu.VMEM((1, _D), _f32)],
)


def kernel(h, edge_index, e, snorm_n, snorm_e, W_emb, b_emb, Ws, bs,
           gammas, betas, W_mlp1, b_mlp1, W_mlp2, b_mlp2):
    del e, snorm_e
    src3 = edge_index[0].reshape(_NW, _NCHUNK, _CH)
    dst3 = edge_index[1].reshape(_NW, _NCHUNK, _CH)
    src2 = edge_index[0].reshape(_NW, _EPW)
    znd = jnp.zeros((_NP, _D), _f32)
    i80 = jnp.arange(80, dtype=jnp.int32).reshape(1, 80)

    _sc_degrees, _sc_aggregate = _sc_kernels()
    srcf = edge_index[0].reshape(_NW, _EPW)
    dstf = edge_index[1].reshape(_NW, _EPW)
    dd = _sc_degrees(srcf, dstf, znd, i80)
    deg_o = (dd[0:80] + dd[160:240]).reshape(-1)[:_N, None]
    deg_i = (dd[80:160] + dd[240:320]).reshape(-1)[:_N, None]
    x, t, nout, ninsn, snb = _embed(h, W_emb, b_emb.reshape(1, _D),
                                    deg_o, deg_i, snorm_n, Ws[0])
    n_layers = len(Ws)
    for i in range(n_layers):
        p = _sc_aggregate(t, src2, dst3, znd)
        b2 = bs[i].reshape(1, _D)
        g2 = gammas[i].reshape(1, _D)
        bt2 = betas[i].reshape(1, _D)
        if i < n_layers - 1:
            x, t = _layer_mid(p[:_N], p[_NP:_NP + _N], ninsn, snb, b2, g2,
                              bt2, x, nout, Ws[i + 1])
        else:
            (out,) = _layer_last(p[:_N], p[_NP:_NP + _N], ninsn, snb, b2, g2,
                                 bt2, x, W_mlp1, b_mlp1.reshape(1, _D),
                                 W_mlp2, b_mlp2.reshape(1, 10))
    return out


# speedup vs baseline: 9.9607x; 1.0011x over previous
import functools

import jax
import jax.numpy as jnp
from jax import lax
from jax.experimental import pallas as pl
from jax.experimental.pallas import tpu as pltpu
from jax.experimental.pallas import tpu_sc as plsc

_N = 10000
_E = 320000
_D = 128
_NC = 2
_NS = 16
_NW = _NC * _NS
_EPW = _E // _NW
_CH = 80
_NCHUNK = _EPW // _CH
_NP = 10240
_RPT = _NP // _NS
_BN = 2000
_G = _N // _BN

_f32 = jnp.float32


def _sc_degrees_body(src_hbm, dst_hbm, znd_hbm, i80_hbm, out_hbm,
                     e1d, ta, tb, i80, tsha, tshb):
    c = lax.axis_index("c")
    s = lax.axis_index("s")
    wid = s * _NC + c
    zv = jnp.zeros((16,), _f32)
    ones = jnp.full((16,), 1.0, _f32)
    pltpu.sync_copy(i80_hbm, i80)

    @pl.when(s == 0)
    def _():
        pltpu.sync_copy(znd_hbm.at[pl.ds(0, 80)], tsha)
        pltpu.sync_copy(znd_hbm.at[pl.ds(0, 80)], tshb)

    @pl.loop(0, 80)
    def _(r):
        @pl.loop(0, 8)
        def _(k):
            ta[r, pl.ds(k * 16, 16)] = zv
            tb[r, pl.ds(k * 16, 16)] = zv

    pltpu.sync_copy(src_hbm.at[wid], e1d)

    @pl.loop(0, _EPW // 16)
    def _(j):
        v = e1d[pl.ds(j * 16, 16)]
        plsc.addupdate_scatter(ta, [jnp.right_shift(v, 7),
                                    jnp.bitwise_and(v, 127)], ones)

    pltpu.sync_copy(dst_hbm.at[wid], e1d)

    @pl.loop(0, _EPW // 16)
    def _(j):
        v = e1d[pl.ds(j * 16, 16)]
        plsc.addupdate_scatter(tb, [jnp.right_shift(v, 7),
                                    jnp.bitwise_and(v, 127)], ones)

    plsc.subcore_barrier()
    pltpu.sync_copy(ta, tsha.at[i80.at[0]], add=True)
    pltpu.sync_copy(tb, tshb.at[i80.at[0]], add=True)
    plsc.subcore_barrier()

    @pl.when(s == 0)
    def _():
        pltpu.sync_copy(tsha, out_hbm.at[pl.ds(c * 160, 80)])

    @pl.when(s == 1)
    def _():
        pltpu.sync_copy(tshb, out_hbm.at[pl.ds(c * 160 + 80, 80)])


def _sc_aggregate_body(t_hbm, src_hbm, dst_hbm, znd_hbm, out_hbm,
                       sidx, didx, r0, r1, acc, sg0, sg1):
    c = lax.axis_index("c")
    s = lax.axis_index("s")
    wid = s * _NC + c
    my = pl.ds(s * _RPT, _RPT)
    pltpu.sync_copy(src_hbm.at[wid], sidx)
    pltpu.sync_copy(dst_hbm.at[wid], didx)
    pltpu.sync_copy(znd_hbm.at[my], acc.at[my])
    plsc.subcore_barrier()

    def gstart(i, buf, sem):
        pltpu.async_copy(t_hbm.at[sidx.at[pl.ds(i * _CH, _CH)]], buf, sem)

    def gwait(buf, sem):
        pltpu.make_async_copy(t_hbm.at[pl.ds(0, _CH)], buf, sem).wait()

    def sadd(i, buf):
        pltpu.sync_copy(buf, acc.at[didx.at[i]], add=True)

    gstart(0, r0, sg0)
    gstart(1, r1, sg1)

    @pl.loop(0, (_NCHUNK - 3) // 2)
    def _(j):
        i = 2 * j
        gwait(r0, sg0)
        sadd(i, r0)
        gstart(i + 2, r0, sg0)
        gwait(r1, sg1)
        sadd(i + 1, r1)
        gstart(i + 3, r1, sg1)

    gwait(r0, sg0)
    sadd(_NCHUNK - 3, r0)
    gstart(_NCHUNK - 1, r0, sg0)
    gwait(r1, sg1)
    sadd(_NCHUNK - 2, r1)
    gwait(r0, sg0)
    sadd(_NCHUNK - 1, r0)

    plsc.subcore_barrier()
    pltpu.sync_copy(acc.at[my], out_hbm.at[pl.ds(c * _NP + s * _RPT, _RPT)])


@functools.cache
def _sc_kernels():
    import dataclasses
    mesh = plsc.VectorSubcoreMesh(core_axis_name="c", subcore_axis_name="s",
                                  num_cores=_NC, num_subcores=_NS)
    cp = pltpu.CompilerParams()
    if "needs_layout_passes" in pltpu.CompilerParams.__dataclass_fields__:
        cp = dataclasses.replace(cp, needs_layout_passes=False)
    degrees = pl.kernel(
        _sc_degrees_body,
        mesh=mesh,
        out_type=jax.ShapeDtypeStruct((_NC * 160, _D), _f32),
        scratch_types=[pltpu.VMEM((_EPW,), jnp.int32),
                       pltpu.VMEM((80, _D), _f32),
                       pltpu.VMEM((80, _D), _f32),
                       pltpu.VMEM((1, 80), jnp.int32),
                       pltpu.VMEM_SHARED((80, _D), _f32),
                       pltpu.VMEM_SHARED((80, _D), _f32)],
        compiler_params=cp,
    )
    aggregate = pl.kernel(
        _sc_aggregate_body,
        mesh=mesh,
        out_type=jax.ShapeDtypeStruct((_NC * _NP, _D), _f32),
        scratch_types=[pltpu.VMEM((_EPW,), jnp.int32),
                       pltpu.VMEM((_NCHUNK, _CH), jnp.int32),
                       pltpu.VMEM((_CH, _D), _f32),
                       pltpu.VMEM((_CH, _D), _f32),
                       pltpu.VMEM_SHARED((_NP, _D), _f32),
                       pltpu.SemaphoreType.DMA,
                       pltpu.SemaphoreType.DMA],
    )
    return degrees, aggregate


def _row_spec():
    return pl.BlockSpec((_BN, _D), lambda i: (i, 0))


def _full_spec(shape):
    return pl.BlockSpec(shape, lambda i: tuple(0 for _ in shape))


def _embed_body(h, we, be, d0, d1, sn, w1, x, t, nout, ninsn, snb):
    deg_o = d0[...]
    deg_i = d1[...]
    no = jnp.where(deg_o > 0, lax.rsqrt(deg_o), 0.0)
    ni = jnp.where(deg_i > 0, lax.rsqrt(deg_i), 0.0)
    s = sn[...]
    nob = jnp.broadcast_to(no, (_BN, _D))
    nout[...] = nob
    ninsn[...] = jnp.broadcast_to(ni * s, (_BN, _D))
    snb[...] = jnp.broadcast_to(s, (_BN, _D))
    xv = jnp.dot(h[...], we[...], preferred_element_type=_f32) + be[...]
    x[...] = xv
    t[...] = jnp.dot(xv * nob, w1[...], preferred_element_type=_f32)


_embed = pl.pallas_call(
    _embed_body,
    grid=(_G,),
    in_specs=[_row_spec(), _full_spec((_D, _D)), _full_spec((1, _D)),
              pl.BlockSpec((_BN, 1), lambda i: (i, 0)),
              pl.BlockSpec((_BN, 1), lambda i: (i, 0)),
              pl.BlockSpec((_BN, 1), lambda i: (i, 0)),
              _full_spec((_D, _D))],
    out_specs=[_row_spec()] * 5,
    out_shape=[jax.ShapeDtypeStruct((_N, _D), _f32)] * 5,
)


def _stats_phase(p0, p1, ninsn, snb, b, u_s, st_s, i):
    uv = (p0[...] + p1[...]) * ninsn[...] + b[...] * snb[...]
    u_s[pl.ds(i * _BN, _BN), :] = uv

    @pl.when(i == 0)
    def _():
        st_s[...] = jnp.zeros((2, _D), _f32)

    st_s[0:1, :] += jnp.sum(uv, axis=0, keepdims=True)
    st_s[1:2, :] += jnp.sum(uv * uv, axis=0, keepdims=True)


def _apply_phase(g, bt, xp, u_s, st_s, i):
    mean = st_s[0:1, :] * (1.0 / _N)
    var = st_s[1:2, :] * (1.0 / _N) - mean * mean
    rstd = lax.rsqrt(var + 1e-5)
    uv = u_s[pl.ds(i * _BN, _BN), :]
    return jax.nn.relu((uv - mean) * rstd * g[...] + bt[...]) + xp[...]


def _layer_mid_body(p0, p1, ninsn, snb, b, g, bt, xp, nout, wn, x, t, u_s, st_s):
    ph = pl.program_id(0)
    i = pl.program_id(1)

    @pl.when(ph == 0)
    def _():
        _stats_phase(p0, p1, ninsn, snb, b, u_s, st_s, i)

    @pl.when(ph == 1)
    def _():
        xv = _apply_phase(g, bt, xp, u_s, st_s, i)
        x[...] = xv
        t[...] = jnp.dot(xv * nout[...], wn[...], preferred_element_type=_f32)


def _p0_spec():
    return pl.BlockSpec((_BN, _D), lambda ph, i: ((1 - ph) * i, 0))


def _p1_spec():
    return pl.BlockSpec((_BN, _D), lambda ph, i: (ph * i, 0))


def _c_spec(shape):
    return pl.BlockSpec(shape, lambda ph, i: tuple(0 for _ in shape))


_layer_mid = pl.pallas_call(
    _layer_mid_body,
    grid=(2, _G),
    in_specs=[_p0_spec(), _p0_spec(), _p0_spec(), _p0_spec(),
              _c_spec((1, _D)), _c_spec((1, _D)), _c_spec((1, _D)),
              _p1_spec(), _p1_spec(), _c_spec((_D, _D))],
    out_specs=[_p1_spec(), _p1_spec()],
    out_shape=[jax.ShapeDtypeStruct((_N, _D), _f32)] * 2,
    scratch_shapes=[pltpu.VMEM((_N, _D), _f32), pltpu.VMEM((2, _D), _f32)],
)


def _layer_last_body(p0, p1, ninsn, snb, b, g, bt, xp, w1, b1, w2, b2, o,
                     u_s, st_s, xs_s):
    ph = pl.program_id(0)
    i = pl.program_id(1)

    @pl.when(ph == 0)
    def _():
        _stats_phase(p0, p1, ninsn, snb, b, u_s, st_s, i)

    @pl.when(ph == 1)
    def _():
        xv = _apply_phase(g, bt, xp, u_s, st_s, i)

        @pl.when(i == 0)
        def _():
            xs_s[...] = jnp.zeros((1, _D), _f32)

        xs_s[...] += jnp.sum(xv, axis=0, keepdims=True)

        @pl.when(i == _G - 1)
        def _():
            hg = xs_s[...] * (1.0 / _N)
            z = jax.nn.relu(jnp.dot(hg, w1[...], preferred_element_type=_f32)
                            + b1[...])
            o[...] = jnp.dot(z, w2[...], preferred_element_type=_f32) + b2[...]


_layer_last = pl.pallas_call(
    _layer_last_body,
    grid=(2, _G),
    in_specs=[_p0_spec(), _p0_spec(), _p0_spec(), _p0_spec(),
              _c_spec((1, _D)), _c_spec((1, _D)), _c_spec((1, _D)),
              _p1_spec(), _c_spec((_D, _D)), _c_spec((1, _D)),
              _c_spec((_D, 10)), _c_spec((1, 10))],
    out_specs=[_c_spec((1, 10))],
    out_shape=[jax.ShapeDtypeStruct((1, 10), _f32)],
    scratch_shapes=[pltpu.VMEM((_N, _D), _f32), pltpu.VMEM((2, _D), _f32),
                    pltpu.VMEM((1, _D), _f32)],
)


def kernel(h, edge_index, e, snorm_n, snorm_e, W_emb, b_emb, Ws, bs,
           gammas, betas, W_mlp1, b_mlp1, W_mlp2, b_mlp2):
    del e, snorm_e
    dst3 = edge_index[1].reshape(_NW, _NCHUNK, _CH)
    src2 = edge_index[0].reshape(_NW, _EPW)
    dstf = edge_index[1].reshape(_NW, _EPW)
    znd = jnp.zeros((_NP, _D), _f32)
    i80 = jnp.arange(80, dtype=jnp.int32).reshape(1, 80)

    _sc_degrees, _sc_aggregate = _sc_kernels()
    dd = _sc_degrees(src2, dstf, znd, i80)
    deg_o = (dd[0:80] + dd[160:240]).reshape(-1)[:_N, None]
    deg_i = (dd[80:160] + dd[240:320]).reshape(-1)[:_N, None]
    x, t, nout, ninsn, snb = _embed(h, W_emb, b_emb.reshape(1, _D),
                                    deg_o, deg_i, snorm_n, Ws[0])
    n_layers = len(Ws)
    for i in range(n_layers):
        p = _sc_aggregate(t, src2, dst3, znd)
        b2 = bs[i].reshape(1, _D)
        g2 = gammas[i].reshape(1, _D)
        bt2 = betas[i].reshape(1, _D)
        if i < n_layers - 1:
            x, t = _layer_mid(p[:_N], p[_NP:_NP + _N], ninsn, snb, b2, g2,
                              bt2, x, nout, Ws[i + 1])
        else:
            (out,) = _layer_last(p[:_N], p[_NP:_NP + _N], ninsn, snb, b2, g2,
                                 bt2, x, W_mlp1, b_mlp1.reshape(1, _D),
                                 W_mlp2, b_mlp2.reshape(1, 10))
    return out
